# Initial kernel scaffold; baseline (speedup 1.0000x reference)
import jax
import jax.numpy as jnp
from jax.experimental import pallas as pl


def kernel(x, node_id, neighbor_idx, interact_score, initial_score, keep_rate, W):
    raise NotImplementedError("write your pallas kernel here")


# speedup vs baseline: 6.2013x; 6.2013x over previous
import functools

import jax
import jax.numpy as jnp
from jax import lax
from jax.experimental import pallas as pl
from jax.experimental.pallas import tpu as pltpu
from jax.experimental.pallas import tpu_sc as plsc

NN = 10000
DD = 32
FF = 128
KK = 8
LL = 16
NW = 32
BPW = 320
NPAD = NW * BPW
NEG_INF = float("-inf")


def _sc_body(xp, nbrp, sco, ini, krv, nid, agg, selo,
             nbr_v, sco_v, ini_v, w_v, sel_v, kr_v, nid_v, row_v, out_v,
             sem_in, sem_g0, sem_g1, sem_out):
    wid = lax.axis_index("s") * 2 + lax.axis_index("c")
    base = wid * BPW

    c1 = pltpu.async_copy(nbrp.at[pl.ds(base * DD, BPW * DD)], nbr_v, sem_in)
    c2 = pltpu.async_copy(sco.at[wid], sco_v, sem_in)
    c3 = pltpu.async_copy(ini.at[wid], ini_v, sem_in)
    c4 = pltpu.async_copy(krv, kr_v, sem_in)
    c5 = pltpu.async_copy(xp.at[pl.ds(base, BPW)], out_v, sem_in)
    c6 = pltpu.async_copy(nid, nid_v, sem_in)
    c1.wait(); c2.wait(); c3.wait(); c4.wait(); c5.wait(); c6.wait()

    krvec = kr_v[pl.ds(0, LL)]
    kr = krvec[0]
    kr1 = 1.0 - kr

    def group_step(g, _):
        col = pl.multiple_of(g * LL, LL)
        s = [sco_v[d, pl.ds(col, LL)] for d in range(DD)]
        cur = list(s)
        dsel = []
        for _k in range(KK):
            m = cur[0]
            for d in range(1, DD):
                m = jnp.maximum(m, cur[d])
            idx = jnp.full((LL,), DD + 1, jnp.int32)
            for d in range(DD):
                idx = jnp.minimum(
                    idx, jnp.where(cur[d] == m, jnp.int32(d), jnp.int32(DD + 1)))
            dsel.append(idx)
            for d in range(DD):
                cur[d] = jnp.where(idx == d, NEG_INF, cur[d])
        rows = col + lax.iota(jnp.int32, LL)
        coef = jnp.full((LL,), 1.0, jnp.float32)
        for d in range(DD):
            selm = jnp.where(cur[d] == NEG_INF, 1.0, 0.0)
            sig = 1.0 / (1.0 + jnp.exp(-s[d]))
            w = kr * ini_v[d, pl.ds(col, LL)] + kr1 * sig * selm
            plsc.store_scatter(w_v, [rows * (3 * LL) + d], w)
            coef = coef + w
        plsc.store_scatter(w_v, [rows * (3 * LL) + DD], 1.0 / coef)
        for k in range(KK):
            raw = plsc.load_gather(nbr_v, [rows * DD + dsel[k]])
            sid = plsc.load_gather(nid_v, [raw])
            plsc.store_scatter(sel_v, [rows * KK + k], sid)
        return 0

    lax.fori_loop(0, BPW // LL, group_step, 0)

    sems = (sem_g0, sem_g1)

    def issue(i, b):
        off = pl.multiple_of(i * DD, DD)
        return pltpu.async_copy(
            xp.at[nbr_v.at[pl.ds(off, DD)]], row_v.at[b], sems[b])

    issue(0, 0)

    def node_pair(gp, _):
        for b in range(2):
            i = gp * 2 + b
            nxt = 1 - b

            @pl.when(i + 1 < BPW)
            def _():
                issue(i + 1, nxt)

            off = pl.multiple_of(i * DD, DD)
            pltpu.make_async_copy(
                xp.at[nbr_v.at[pl.ds(off, DD)]], row_v.at[b], sems[b]).wait()

            wrow = pl.multiple_of(i * (3 * LL), 3 * LL)
            wa = w_v[pl.ds(wrow, LL)]
            wb = w_v[pl.ds(wrow + LL, LL)]
            wc = w_v[pl.ds(wrow + 2 * LL, LL)]
            acc = [out_v[i, pl.ds(c * LL, LL)] for c in range(FF // LL)]
            for d in range(DD):
                ws = wa[d] if d < LL else wb[d - LL]
                for c in range(FF // LL):
                    acc[c] = acc[c] + ws * row_v[b, d, pl.ds(c * LL, LL)]
            inv = wc[0]
            for c in range(FF // LL):
                out_v[i, pl.ds(c * LL, LL)] = acc[c] * inv
        return 0

    lax.fori_loop(0, BPW // 2, node_pair, 0)

    co = pltpu.async_copy(out_v, agg.at[pl.ds(base, BPW)], sem_out)
    cs = pltpu.async_copy(sel_v, selo.at[pl.ds(base * KK, BPW * KK)], sem_out)
    co.wait()
    cs.wait()


@jax.jit
def kernel(x, node_id, neighbor_idx, interact_score, initial_score, keep_rate, W):
    del W
    xp = jnp.zeros((NPAD, FF), jnp.float32).at[:NN].set(x)
    nbrp = jnp.zeros((NPAD, DD), jnp.int32).at[:NN].set(neighbor_idx).reshape(-1)
    sco = (jnp.zeros((NPAD, DD), jnp.float32).at[:NN].set(interact_score)
           .reshape(NW, BPW, DD).transpose(0, 2, 1))
    ini = (jnp.zeros((NPAD, DD), jnp.float32).at[:NN].set(initial_score)
           .reshape(NW, BPW, DD).transpose(0, 2, 1))
    krv = jnp.broadcast_to(keep_rate.astype(jnp.float32), (LL,))
    nid = jnp.zeros((NPAD,), jnp.int32).at[:NN].set(node_id)

    f = pl.kernel(
        _sc_body,
        out_type=(
            jax.ShapeDtypeStruct((NPAD, FF), jnp.float32),
            jax.ShapeDtypeStruct((NPAD * KK,), jnp.int32),
        ),
        mesh=plsc.VectorSubcoreMesh(core_axis_name="c", subcore_axis_name="s"),
        compiler_params=pltpu.CompilerParams(needs_layout_passes=False),
        scratch_types=[
            pltpu.VMEM((BPW * DD,), jnp.int32),
            pltpu.VMEM((DD, BPW), jnp.float32),
            pltpu.VMEM((DD, BPW), jnp.float32),
            pltpu.VMEM((BPW * 3 * LL,), jnp.float32),
            pltpu.VMEM((BPW * KK,), jnp.int32),
            pltpu.VMEM((LL,), jnp.float32),
            pltpu.VMEM((NPAD,), jnp.int32),
            pltpu.VMEM((2, DD, FF), jnp.float32),
            pltpu.VMEM((BPW, FF), jnp.float32),
            pltpu.SemaphoreType.DMA,
            pltpu.SemaphoreType.DMA,
            pltpu.SemaphoreType.DMA,
            pltpu.SemaphoreType.DMA,
        ],
    )
    agg, sel = f(xp, nbrp, sco, ini, krv, nid)
    return agg[:NN], sel.reshape(NPAD, KK)[:NN]
